# baseline (speedup 1.0000x reference)
import functools

import jax
import jax.numpy as jnp
from jax import lax
from jax.experimental import pallas as pl
from jax.experimental.pallas import tpu as pltpu
from jax.experimental.pallas import tpu_sc as plsc

_B, _C1, _H, _W = 8, 384, 64, 64
_HW = _H * _W
_C = 2 * _C1
_K = 256
_CCHUNK = 128
_NCHUNK = _C1 // _CCHUNK
_RCHUNK = 128

_NW = 32
_RPW = (_B * _K) // _NW
_GCH = 4


def _pool_sort_kernel(x1_ref, x2_ref, idx_ref, tot_ref, pooled_ref):
    ci = pl.program_id(1)
    x1 = x1_ref[0]
    x2 = x2_ref[0]

    def _chansum(x):
        s3 = jnp.sum(x.reshape(_CCHUNK, _HW // 128, 128), axis=1)
        return jnp.sum(s3.T, axis=0)

    pooled_ref[0, pl.ds(ci * _CCHUNK, _CCHUNK)] = _chansum(x1)
    pooled_ref[0, pl.ds(_C1 + ci * _CCHUNK, _CCHUNK)] = _chansum(x2)

    part = jnp.sum(x1, axis=0) + jnp.sum(x2, axis=0)

    @pl.when(ci == 0)
    def _init():
        tot_ref[0, 0] = part

    @pl.when(ci > 0)
    def _acc():
        tot_ref[0, 0] += part

    @pl.when(ci == _NCHUNK - 1)
    def _sort():
        nb = _C // 128
        inv = 1.0 / _HW
        pch = [pooled_ref[0, k * 128:(k + 1) * 128][None, :] * inv
               for k in range(nb)]
        io_sub = jax.lax.broadcasted_iota(jnp.int32, (128, 128), 0)
        io_lane = jax.lax.broadcasted_iota(jnp.int32, (128, 128), 1)
        io_sub_f = io_sub.astype(jnp.float32)
        io_lane_f = io_lane.astype(jnp.float32)

        rank_rows = []
        for a in range(nb):
            vc_a = jnp.broadcast_to(pch[a], (128, 128)).T
            row_g = 128 * a + io_sub
            acc = jnp.zeros((128, 128), jnp.float32)
            for k in range(nb):
                vr_k = jnp.broadcast_to(pch[k], (128, 128))
                col_g = 128 * k + io_lane
                m = (vr_k > vc_a) | ((vr_k == vc_a) & (col_g < row_g))
                acc += jnp.where(m, 1.0, 0.0)
            rank_rows.append(jnp.sum(acc.T, axis=0)[None, :])

        for jb in range(_K // 128):
            jv = 128.0 * jb + io_sub_f
            acc2 = jnp.zeros((128, 128), jnp.float32)
            for k in range(nb):
                rk = jnp.broadcast_to(rank_rows[k], (128, 128))
                col_g = 128.0 * k + io_lane_f
                acc2 += jnp.where(rk == jv, col_g, 0.0)
            idx_b = jnp.sum(acc2.T, axis=0).astype(jnp.int32)
            idx_ref[0, 0, pl.ds(jb * 128, 128)] = idx_b


def _sc_gather_kernel(y1_ref, y2_ref, r1_ref, r2_ref, sel_ref, out_ref,
                      r1_v, r2_v, sel_v, buf1, buf2,
                      gs1a, gs1b, gs2a, gs2b, ssem):
    gs1 = (gs1a, gs1b)
    gs2 = (gs2a, gs2b)
    wid = lax.axis_index("s") * 2 + lax.axis_index("c")
    base = wid * _RPW
    pltpu.sync_copy(r1_ref.at[wid], r1_v)
    pltpu.sync_copy(r2_ref.at[wid], r2_v)
    pltpu.sync_copy(sel_ref.at[pl.ds(base, _RPW)], sel_v)

    nq = _RPW // _GCH

    def _start(q):
        s = q % 2
        g1 = pltpu.async_copy(
            y1_ref.at[r1_v.at[q]], buf1.at[s], gs1[s])
        g2 = pltpu.async_copy(
            y2_ref.at[r2_v.at[q]], buf2.at[s], gs2[s])
        return g1, g2

    def _drain_stores(q):
        pltpu.make_async_copy(
            y1_ref.at[pl.ds(0, _GCH)],
            out_ref.at[pl.ds(base + q * _GCH, _GCH)], ssem).wait()

    pend = _start(0)
    for q in range(nq):
        s = q % 2
        nxt = None
        if q + 1 < nq:
            if q >= 1:
                _drain_stores(q - 1)
            nxt = _start(q + 1)
        g1, g2 = pend
        g1.wait()
        g2.wait()
        for i in range(_GCH):
            row = q * _GCH + i
            win = (row // 16) * 16
            mv = sel_v[pl.ds(win, 16)]
            sc = mv[row - win]

            @pl.when(sc > 0.5)
            def _from1(i=i, row=row, s=s):
                pltpu.async_copy(buf1.at[s, i], out_ref.at[base + row], ssem)

            @pl.when(sc <= 0.5)
            def _from2(i=i, row=row, s=s):
                pltpu.async_copy(buf2.at[s, i], out_ref.at[base + row], ssem)
        pend = nxt
    _drain_stores(nq - 2)
    _drain_stores(nq - 1)


def _fix_kernel(out_ref, tot_ref, fixed_ref, acc_ref):
    ci = pl.program_id(1)
    x = out_ref[0]
    grow = ci * 64 + jax.lax.broadcasted_iota(jnp.int32, (64, 1), 0)
    part = jnp.sum(jnp.where(grow < _K - 1, x, 0.0), axis=0)

    @pl.when(ci == 0)
    def _init():
        acc_ref[...] = part[None]

    @pl.when(ci > 0)
    def _acc():
        acc_ref[...] += part[None]

    @pl.when(ci == _K // 64 - 1)
    def _fix():
        fixed_ref[0, 0] = tot_ref[0, 0] - acc_ref[0]


def kernel(x1, x2):
    y1 = x1.reshape(_B, _C1, _HW)
    y2 = x2.reshape(_B, _C1, _HW)

    idx, tot = pl.pallas_call(
        _pool_sort_kernel,
        grid=(_B, _NCHUNK),
        in_specs=[
            pl.BlockSpec((1, _CCHUNK, _HW), lambda b, c: (b, c, 0)),
            pl.BlockSpec((1, _CCHUNK, _HW), lambda b, c: (b, c, 0)),
        ],
        out_specs=[
            pl.BlockSpec((1, 1, _K), lambda b, c: (b, 0, 0)),
            pl.BlockSpec((1, 1, _HW), lambda b, c: (b, 0, 0)),
        ],
        out_shape=[
            jax.ShapeDtypeStruct((_B, 1, _K), jnp.int32),
            jax.ShapeDtypeStruct((_B, 1, _HW), jnp.float32),
        ],
        scratch_shapes=[pltpu.VMEM((1, _C), jnp.float32)],
        compiler_params=pltpu.CompilerParams(
            dimension_semantics=("arbitrary", "arbitrary")),
    )(y1, y2)

    cflat = idx.reshape(_B * _K)
    bb = jax.lax.broadcasted_iota(jnp.int32, (_B * _K,), 0) // _K
    r1 = bb * _C1 + jnp.clip(cflat, 0, _C1 - 1)
    r2 = bb * _C1 + jnp.clip(cflat - _C1, 0, _C1 - 1)
    sel = (cflat < _C1).astype(jnp.float32)

    r1 = r1.reshape(_NW, _RPW // _GCH, _GCH)
    r2 = r2.reshape(_NW, _RPW // _GCH, _GCH)

    t1 = x1.reshape(_B * _C1, _HW)
    t2 = x2.reshape(_B * _C1, _HW)

    mesh = plsc.VectorSubcoreMesh(core_axis_name="c", subcore_axis_name="s")
    gathered = pl.kernel(
        _sc_gather_kernel,
        mesh=mesh,
        out_type=jax.ShapeDtypeStruct((_B * _K, _HW), jnp.float32),
        scratch_types=[
            pltpu.VMEM((_RPW // _GCH, _GCH), jnp.int32),
            pltpu.VMEM((_RPW // _GCH, _GCH), jnp.int32),
            pltpu.VMEM((_RPW,), jnp.float32),
            pltpu.VMEM((2, _GCH, _HW), jnp.float32),
            pltpu.VMEM((2, _GCH, _HW), jnp.float32),
            pltpu.SemaphoreType.DMA,
            pltpu.SemaphoreType.DMA,
            pltpu.SemaphoreType.DMA,
            pltpu.SemaphoreType.DMA,
            pltpu.SemaphoreType.DMA,
        ],
    )(t1, t2, r1, r2, sel)

    out3 = gathered.reshape(_B, _K, _HW)
    fixed = pl.pallas_call(
        _fix_kernel,
        grid=(_B, _K // 64),
        in_specs=[
            pl.BlockSpec((1, 64, _HW), lambda b, c: (b, c, 0)),
            pl.BlockSpec((1, 1, _HW), lambda b, c: (b, 0, 0)),
        ],
        out_specs=pl.BlockSpec((1, 1, _HW), lambda b, c: (b, 0, 0)),
        out_shape=jax.ShapeDtypeStruct((_B, 1, _HW), jnp.float32),
        scratch_shapes=[pltpu.VMEM((1, _HW), jnp.float32)],
        compiler_params=pltpu.CompilerParams(
            dimension_semantics=("arbitrary", "arbitrary")),
    )(out3, tot)

    out3 = jax.lax.dynamic_update_slice(out3, fixed, (0, _K - 1, 0))
    return out3.reshape(_B, _K, _H, _W)
